# Initial kernel scaffold; baseline (speedup 1.0000x reference)
import jax
import jax.numpy as jnp
from jax.experimental import pallas as pl


def kernel(x, edge_index, edge_length, edge_attr, Wf1, bf1, Wf2, bf2, Wl1, Wl2, bl2, Wl, bl):
    raise NotImplementedError("write your pallas kernel here")


# speedup vs baseline: 1.1685x; 1.1685x over previous
import functools

import jax
import jax.numpy as jnp
from jax import lax
from jax.experimental import pallas as pl
from jax.experimental.pallas import tpu as pltpu
from jax.experimental.pallas import tpu_sc as plsc

N = 10000
E = 320000
H = 128
F = 128
G = 16
HH = 64
CUTOFF = 10.0

NC = 2
NS = 16
EPT = E // NS
CHUNK = 80
NCHUNK = EPT // CHUNK
NPAD = 10240
RPT = NPAD // NS


def _x1_body(x_ref, w_ref, oa_ref, ob_ref):
    x1 = jnp.dot(x_ref[...], w_ref[...], preferred_element_type=jnp.float32)
    oa_ref[...] = x1[:, :HH]
    ob_ref[...] = x1[:, HH:]


def _tc_x1(x, Wl1):
    return pl.pallas_call(
        _x1_body,
        grid=(10,),
        in_specs=[
            pl.BlockSpec((N // 10, H), lambda i: (i, 0)),
            pl.BlockSpec((H, F), lambda i: (0, 0)),
        ],
        out_specs=[
            pl.BlockSpec((N // 10, HH), lambda i: (i, 0)),
            pl.BlockSpec((N // 10, HH), lambda i: (i, 0)),
        ],
        out_shape=[
            jax.ShapeDtypeStruct((N, HH), jnp.float32),
            jax.ShapeDtypeStruct((N, HH), jnp.float32),
        ],
    )(x, Wl1)


def _filt_body(ea_ref, el_ref, wf1_ref, bf1_ref, wf2_ref, bf2_ref, oa_ref, ob_ref):
    h = jax.nn.gelu(
        jnp.dot(ea_ref[...], wf1_ref[...], preferred_element_type=jnp.float32)
        + bf1_ref[...]
    )
    w = jnp.dot(h, wf2_ref[...], preferred_element_type=jnp.float32) + bf2_ref[...]
    el = el_ref[...]
    c = 0.5 * (jnp.cos(el * (jnp.pi / CUTOFF)) + 1.0)
    c = c * (el <= CUTOFF).astype(jnp.float32) * (el >= 0.0).astype(jnp.float32)
    w = w * c
    oa_ref[...] = w[:, :HH]
    ob_ref[...] = w[:, HH:]


_BE = 2000


def _tc_filter(edge_attr, edge_length2d, Wf1, bf1, Wf2, bf2):
    nb = E // _BE
    return pl.pallas_call(
        _filt_body,
        grid=(nb,),
        in_specs=[
            pl.BlockSpec((_BE, G), lambda i: (i, 0)),
            pl.BlockSpec((_BE, 1), lambda i: (i, 0)),
            pl.BlockSpec((G, F), lambda i: (0, 0)),
            pl.BlockSpec((1, F), lambda i: (0, 0)),
            pl.BlockSpec((F, F), lambda i: (0, 0)),
            pl.BlockSpec((1, F), lambda i: (0, 0)),
        ],
        out_specs=[
            pl.BlockSpec((_BE, HH), lambda i: (i, 0)),
            pl.BlockSpec((_BE, HH), lambda i: (i, 0)),
        ],
        out_shape=[
            jax.ShapeDtypeStruct((E, HH), jnp.float32),
            jax.ShapeDtypeStruct((E, HH), jnp.float32),
        ],
    )(edge_attr, edge_length2d, Wf1, bf1, Wf2, bf2)


def _sc_body(x1a_hbm, x1b_hbm, wfa_hbm, wfb_hbm, src_hbm, dst_hbm,
             outa_hbm, outb_hbm,
             src_v, dst_v, rows_v, wf_v, zb_v, dump_v, agg_sh, sem):
    cid = lax.axis_index("c")
    sid = lax.axis_index("s")

    zero16 = jnp.zeros((16,), jnp.float32)

    def zrow(i, carry):
        for j in range(HH // 16):
            zb_v[i, pl.ds(j * 16, 16)] = zero16
        return carry

    lax.fori_loop(0, 32, zrow, 0)

    def zcp(k, carry):
        pltpu.sync_copy(zb_v, agg_sh.at[pl.ds(sid * RPT + k * 32, 32)])
        return carry

    lax.fori_loop(0, RPT // 32, zcp, 0)

    pltpu.sync_copy(src_hbm.at[sid], src_v)
    pltpu.sync_copy(dst_hbm.at[sid], dst_v)
    plsc.subcore_barrier()

    ebase = sid * EPT

    def run_half(x1_hbm, wf_hbm):
        def chunk_body(g, carry):
            pltpu.async_copy(x1_hbm.at[src_v.at[g]], rows_v, sem).wait()
            pltpu.sync_copy(wf_hbm.at[pl.ds(ebase + g * CHUNK, CHUNK)], wf_v)

            def mul_row(i, c2):
                for j in range(HH // 16):
                    sl = pl.ds(j * 16, 16)
                    rows_v[i, sl] = rows_v[i, sl] * wf_v[i, sl]
                return c2

            lax.fori_loop(0, CHUNK, mul_row, 0)
            pltpu.sync_copy(rows_v, agg_sh.at[dst_v.at[g]], add=True)
            return carry

        lax.fori_loop(0, NCHUNK, chunk_body, 0)

    @pl.when(cid == 0)
    def _():
        run_half(x1a_hbm, wfa_hbm)

    @pl.when(cid == 1)
    def _():
        run_half(x1b_hbm, wfb_hbm)

    plsc.subcore_barrier()

    def dump_half(out_hbm):
        for k in range(RPT // 128):
            pltpu.sync_copy(agg_sh.at[pl.ds(sid * RPT + k * 128, 128)], dump_v)
            pltpu.sync_copy(dump_v, out_hbm.at[pl.ds(sid * RPT + k * 128, 128)])

    @pl.when(cid == 0)
    def _():
        dump_half(outa_hbm)

    @pl.when(cid == 1)
    def _():
        dump_half(outb_hbm)


def _sc_scatter(x1a, x1b, wfa, wfb, src, dst):
    mesh = plsc.VectorSubcoreMesh(core_axis_name="c", subcore_axis_name="s")
    fn = functools.partial(
        pl.kernel,
        mesh=mesh,
        out_type=[
            jax.ShapeDtypeStruct((NPAD, HH), jnp.float32),
            jax.ShapeDtypeStruct((NPAD, HH), jnp.float32),
        ],
        scratch_types=[
            pltpu.VMEM((NCHUNK, CHUNK), jnp.int32),
            pltpu.VMEM((NCHUNK, CHUNK), jnp.int32),
            pltpu.VMEM((CHUNK, HH), jnp.float32),
            pltpu.VMEM((CHUNK, HH), jnp.float32),
            pltpu.VMEM((32, HH), jnp.float32),
            pltpu.VMEM((128, HH), jnp.float32),
            pltpu.VMEM_SHARED((NPAD, HH), jnp.float32),
            pltpu.SemaphoreType.DMA,
        ],
        compiler_params=pltpu.CompilerParams(use_tc_tiling_on_sc=False),
    )(_sc_body)
    return fn(x1a, x1b, wfa, wfb, src, dst)


def _out_body(pa_ref, pb_ref, wl2_ref, bl2_ref, wl_ref, bl_ref, o_ref):
    z = jnp.concatenate([pa_ref[...], pb_ref[...]], axis=1)
    x2 = jnp.dot(z, wl2_ref[...], preferred_element_type=jnp.float32) + bl2_ref[...]
    x3 = jax.nn.gelu(x2)
    o_ref[...] = jnp.dot(x3, wl_ref[...], preferred_element_type=jnp.float32) + bl_ref[...]


def _tc_out(pa, pb, Wl2, bl2, Wl, bl):
    return pl.pallas_call(
        _out_body,
        grid=(10,),
        in_specs=[
            pl.BlockSpec((N // 10, HH), lambda i: (i, 0)),
            pl.BlockSpec((N // 10, HH), lambda i: (i, 0)),
            pl.BlockSpec((F, H), lambda i: (0, 0)),
            pl.BlockSpec((1, H), lambda i: (0, 0)),
            pl.BlockSpec((H, H), lambda i: (0, 0)),
            pl.BlockSpec((1, H), lambda i: (0, 0)),
        ],
        out_specs=pl.BlockSpec((N // 10, H), lambda i: (i, 0)),
        out_shape=jax.ShapeDtypeStruct((N, H), jnp.float32),
    )(pa, pb, Wl2, bl2, Wl, bl)


def kernel(x, edge_index, edge_length, edge_attr, Wf1, bf1, Wf2, bf2,
           Wl1, Wl2, bl2, Wl, bl):
    x1a, x1b = _tc_x1(x, Wl1)
    wfa, wfb = _tc_filter(edge_attr, edge_length.reshape(E, 1), Wf1,
                          bf1.reshape(1, F), Wf2, bf2.reshape(1, F))
    src = edge_index[0].reshape(NS, NCHUNK, CHUNK)
    dst = edge_index[1].reshape(NS, NCHUNK, CHUNK)
    pa, pb = _sc_scatter(x1a, x1b, wfa, wfb, src, dst)
    out = _tc_out(pa[:N], pb[:N], Wl2, bl2.reshape(1, H),
                  Wl, bl.reshape(1, H))
    return out


# speedup vs baseline: 1.4886x; 1.2739x over previous
import functools

import jax
import jax.numpy as jnp
from jax import lax
from jax.experimental import pallas as pl
from jax.experimental.pallas import tpu as pltpu
from jax.experimental.pallas import tpu_sc as plsc

N = 10000
E = 320000
H = 128
F = 128
G = 16
HH = 64
CUTOFF = 10.0

NC = 2
NS = 16
EPT = E // NS
CHUNK = 80
NCHUNK = EPT // CHUNK
NPAD = 10240
RPT = NPAD // NS


def _x1_body(x_ref, w_ref, oa_ref, ob_ref):
    x1 = jnp.dot(x_ref[...], w_ref[...], preferred_element_type=jnp.float32)
    oa_ref[...] = x1[:, :HH]
    ob_ref[...] = x1[:, HH:]


def _tc_x1(x, Wl1):
    return pl.pallas_call(
        _x1_body,
        grid=(10,),
        in_specs=[
            pl.BlockSpec((N // 10, H), lambda i: (i, 0)),
            pl.BlockSpec((H, F), lambda i: (0, 0)),
        ],
        out_specs=[
            pl.BlockSpec((N // 10, HH), lambda i: (i, 0)),
            pl.BlockSpec((N // 10, HH), lambda i: (i, 0)),
        ],
        out_shape=[
            jax.ShapeDtypeStruct((N, HH), jnp.float32),
            jax.ShapeDtypeStruct((N, HH), jnp.float32),
        ],
    )(x, Wl1)


def _filt_body(ea_ref, el_ref, wf1_ref, bf1_ref, wf2_ref, bf2_ref, oa_ref, ob_ref):
    h = jax.nn.gelu(
        jnp.dot(ea_ref[...], wf1_ref[...], preferred_element_type=jnp.float32)
        + bf1_ref[...]
    )
    w = jnp.dot(h, wf2_ref[...], preferred_element_type=jnp.float32) + bf2_ref[...]
    el = el_ref[...]
    c = 0.5 * (jnp.cos(el * (jnp.pi / CUTOFF)) + 1.0)
    c = c * (el <= CUTOFF).astype(jnp.float32) * (el >= 0.0).astype(jnp.float32)
    w = w * c
    oa_ref[...] = w[:, :HH]
    ob_ref[...] = w[:, HH:]


_BE = 2000


def _tc_filter(edge_attr, edge_length2d, Wf1, bf1, Wf2, bf2):
    nb = E // _BE
    return pl.pallas_call(
        _filt_body,
        grid=(nb,),
        in_specs=[
            pl.BlockSpec((_BE, G), lambda i: (i, 0)),
            pl.BlockSpec((_BE, 1), lambda i: (i, 0)),
            pl.BlockSpec((G, F), lambda i: (0, 0)),
            pl.BlockSpec((1, F), lambda i: (0, 0)),
            pl.BlockSpec((F, F), lambda i: (0, 0)),
            pl.BlockSpec((1, F), lambda i: (0, 0)),
        ],
        out_specs=[
            pl.BlockSpec((_BE, HH), lambda i: (i, 0)),
            pl.BlockSpec((_BE, HH), lambda i: (i, 0)),
        ],
        out_shape=[
            jax.ShapeDtypeStruct((E, HH), jnp.float32),
            jax.ShapeDtypeStruct((E, HH), jnp.float32),
        ],
    )(edge_attr, edge_length2d, Wf1, bf1, Wf2, bf2)


def _sc_body(x1a_hbm, x1b_hbm, wfa_hbm, wfb_hbm, src_hbm, dst_hbm,
             outa_hbm, outb_hbm,
             src_v, dst_v, in_rows, wf_bufs, out_rows, zb_v, dump_v, agg_sh,
             sem_g, sem_w, sem_s):
    cid = lax.axis_index("c")
    sid = lax.axis_index("s")

    zero16 = jnp.zeros((16,), jnp.float32)

    def zrow(i, carry):
        for j in range(HH // 16):
            zb_v[i, pl.ds(j * 16, 16)] = zero16
        return carry

    lax.fori_loop(0, 32, zrow, 0)

    def zcp(k, carry):
        pltpu.sync_copy(zb_v, agg_sh.at[pl.ds(sid * RPT + k * 32, 32)])
        return carry

    lax.fori_loop(0, RPT // 32, zcp, 0)

    pltpu.sync_copy(src_hbm.at[sid], src_v)
    pltpu.sync_copy(dst_hbm.at[sid], dst_v)
    plsc.subcore_barrier()

    ebase = sid * EPT

    def run_half(x1_hbm, wf_hbm):
        def issue_in(g, b):
            pltpu.async_copy(x1_hbm.at[src_v.at[g]], in_rows[b], sem_g[b])
            pltpu.async_copy(wf_hbm.at[pl.ds(ebase + g * CHUNK, CHUNK)],
                             wf_bufs[b], sem_w[b])

        def wait_in(g, b):
            pltpu.make_async_copy(x1_hbm.at[src_v.at[g]], in_rows[b],
                                  sem_g[b]).wait()
            pltpu.make_async_copy(wf_hbm.at[pl.ds(ebase + g * CHUNK, CHUNK)],
                                  wf_bufs[b], sem_w[b]).wait()

        def wait_scatter(g, b):
            pltpu.make_async_copy(out_rows[b], agg_sh.at[dst_v.at[g]],
                                  sem_s[b]).wait()

        issue_in(0, 0)
        issue_in(1, 1)

        def pipe_body(go, carry):
            for b in range(2):
                g = go * 2 + b
                wait_in(g, b)

                @pl.when(g >= 2)
                def _():
                    wait_scatter(g - 2, b)

                def mul_row(i, c2):
                    for j in range(HH // 16):
                        sl = pl.ds(j * 16, 16)
                        out_rows[b][i, sl] = in_rows[b][i, sl] * wf_bufs[b][i, sl]
                    return c2

                lax.fori_loop(0, CHUNK, mul_row, 0)
                pltpu.async_copy(out_rows[b], agg_sh.at[dst_v.at[g]],
                                 sem_s[b], add=True)

                @pl.when(g + 2 < NCHUNK)
                def _():
                    issue_in(g + 2, b)
            return carry

        lax.fori_loop(0, NCHUNK // 2, pipe_body, 0)
        wait_scatter(NCHUNK - 2, 0)
        wait_scatter(NCHUNK - 1, 1)

    @pl.when(cid == 0)
    def _():
        run_half(x1a_hbm, wfa_hbm)

    @pl.when(cid == 1)
    def _():
        run_half(x1b_hbm, wfb_hbm)

    plsc.subcore_barrier()

    def dump_half(out_hbm):
        for k in range(RPT // 128):
            pltpu.sync_copy(agg_sh.at[pl.ds(sid * RPT + k * 128, 128)], dump_v)
            pltpu.sync_copy(dump_v, out_hbm.at[pl.ds(sid * RPT + k * 128, 128)])

    @pl.when(cid == 0)
    def _():
        dump_half(outa_hbm)

    @pl.when(cid == 1)
    def _():
        dump_half(outb_hbm)


def _sc_scatter(x1a, x1b, wfa, wfb, src, dst):
    mesh = plsc.VectorSubcoreMesh(core_axis_name="c", subcore_axis_name="s")
    fn = functools.partial(
        pl.kernel,
        mesh=mesh,
        out_type=[
            jax.ShapeDtypeStruct((NPAD, HH), jnp.float32),
            jax.ShapeDtypeStruct((NPAD, HH), jnp.float32),
        ],
        scratch_types=[
            pltpu.VMEM((NCHUNK, CHUNK), jnp.int32),
            pltpu.VMEM((NCHUNK, CHUNK), jnp.int32),
            [pltpu.VMEM((CHUNK, HH), jnp.float32) for _ in range(2)],
            [pltpu.VMEM((CHUNK, HH), jnp.float32) for _ in range(2)],
            [pltpu.VMEM((CHUNK, HH), jnp.float32) for _ in range(2)],
            pltpu.VMEM((32, HH), jnp.float32),
            pltpu.VMEM((128, HH), jnp.float32),
            pltpu.VMEM_SHARED((NPAD, HH), jnp.float32),
            [pltpu.SemaphoreType.DMA for _ in range(2)],
            [pltpu.SemaphoreType.DMA for _ in range(2)],
            [pltpu.SemaphoreType.DMA for _ in range(2)],
        ],
        compiler_params=pltpu.CompilerParams(use_tc_tiling_on_sc=False),
    )(_sc_body)
    return fn(x1a, x1b, wfa, wfb, src, dst)


def _out_body(pa_ref, pb_ref, wl2_ref, bl2_ref, wl_ref, bl_ref, o_ref):
    z = jnp.concatenate([pa_ref[...], pb_ref[...]], axis=1)
    x2 = jnp.dot(z, wl2_ref[...], preferred_element_type=jnp.float32) + bl2_ref[...]
    x3 = jax.nn.gelu(x2)
    o_ref[...] = jnp.dot(x3, wl_ref[...], preferred_element_type=jnp.float32) + bl_ref[...]


def _tc_out(pa, pb, Wl2, bl2, Wl, bl):
    return pl.pallas_call(
        _out_body,
        grid=(10,),
        in_specs=[
            pl.BlockSpec((N // 10, HH), lambda i: (i, 0)),
            pl.BlockSpec((N // 10, HH), lambda i: (i, 0)),
            pl.BlockSpec((F, H), lambda i: (0, 0)),
            pl.BlockSpec((1, H), lambda i: (0, 0)),
            pl.BlockSpec((H, H), lambda i: (0, 0)),
            pl.BlockSpec((1, H), lambda i: (0, 0)),
        ],
        out_specs=pl.BlockSpec((N // 10, H), lambda i: (i, 0)),
        out_shape=jax.ShapeDtypeStruct((N, H), jnp.float32),
    )(pa, pb, Wl2, bl2, Wl, bl)


def kernel(x, edge_index, edge_length, edge_attr, Wf1, bf1, Wf2, bf2,
           Wl1, Wl2, bl2, Wl, bl):
    x1a, x1b = _tc_x1(x, Wl1)
    wfa, wfb = _tc_filter(edge_attr, edge_length.reshape(E, 1), Wf1,
                          bf1.reshape(1, F), Wf2, bf2.reshape(1, F))
    src = edge_index[0].reshape(NS, NCHUNK, CHUNK)
    dst = edge_index[1].reshape(NS, NCHUNK, CHUNK)
    pa, pb = _sc_scatter(x1a, x1b, wfa, wfb, src, dst)
    out = _tc_out(pa[:N], pb[:N], Wl2, bl2.reshape(1, H),
                  Wl, bl.reshape(1, H))
    return out


# speedup vs baseline: 2.0011x; 1.3443x over previous
import functools

import jax
import jax.numpy as jnp
from jax import lax
from jax.experimental import pallas as pl
from jax.experimental.pallas import tpu as pltpu
from jax.experimental.pallas import tpu_sc as plsc

N = 10000
E = 320000
H = 128
F = 128
G = 16
HH = 64
CUTOFF = 10.0

NC = 2
NS = 16
EPT = E // NS
CHUNK = 80
NCHUNK = EPT // CHUNK
NPHASE = 5
PCHUNK = NCHUNK // NPHASE
NPAD = 10240
RPT = NPAD // NS


def _fast_gelu(x):
    u = 0.7978845608028654 * (x + 0.044715 * x * x * x)
    e = jnp.exp(-2.0 * jnp.abs(u))
    t = (1.0 - e) / (1.0 + e)
    t = jnp.where(u < 0.0, -t, t)
    return 0.5 * x * (1.0 + t)


_COS_C = (0.9999999999193516, -0.4999999988862244, 0.041666664158393055,
          -0.0013888867464020146, 2.4800691210301726e-05,
          -2.75369890919258e-07, 2.0620727253511867e-09,
          -9.774996100930424e-12)


def _cos_poly(t2):
    acc = jnp.full_like(t2, _COS_C[-1])
    for k in range(len(_COS_C) - 2, -1, -1):
        acc = acc * t2 + _COS_C[k]
    return acc


def _x1_body(x_ref, w_ref, oa_ref, ob_ref):
    x1 = jnp.dot(x_ref[...], w_ref[...], preferred_element_type=jnp.float32)
    oa_ref[...] = x1[:, :HH]
    ob_ref[...] = x1[:, HH:]


def _tc_x1(x, Wl1):
    return pl.pallas_call(
        _x1_body,
        grid=(10,),
        in_specs=[
            pl.BlockSpec((N // 10, H), lambda i: (i, 0)),
            pl.BlockSpec((H, F), lambda i: (0, 0)),
        ],
        out_specs=[
            pl.BlockSpec((N // 10, HH), lambda i: (i, 0)),
            pl.BlockSpec((N // 10, HH), lambda i: (i, 0)),
        ],
        out_shape=[
            jax.ShapeDtypeStruct((N, HH), jnp.float32),
            jax.ShapeDtypeStruct((N, HH), jnp.float32),
        ],
    )(x, Wl1)


def _filt_body(ea_ref, el_ref, wf1_ref, bf1_ref, wf2_ref, bf2_ref, oa_ref, ob_ref):
    h = _fast_gelu(
        jnp.dot(ea_ref[...], wf1_ref[...], preferred_element_type=jnp.float32)
        + bf1_ref[...]
    )
    w = jnp.dot(h, wf2_ref[...], preferred_element_type=jnp.float32) + bf2_ref[...]
    el = el_ref[...]
    t = el * (jnp.pi / CUTOFF)
    c = 0.5 * (_cos_poly(t * t) + 1.0)
    c = jnp.where((el <= CUTOFF) & (el >= 0.0), c, 0.0)
    w = w * c
    oa_ref[...] = w[:, :HH]
    ob_ref[...] = w[:, HH:]


_BE = 2000


def _tc_filter(edge_attr, edge_length2d, Wf1, bf1, Wf2, bf2):
    nb = E // _BE
    return pl.pallas_call(
        _filt_body,
        grid=(nb,),
        in_specs=[
            pl.BlockSpec((_BE, G), lambda i: (i, 0)),
            pl.BlockSpec((_BE, 1), lambda i: (i, 0)),
            pl.BlockSpec((G, F), lambda i: (0, 0)),
            pl.BlockSpec((1, F), lambda i: (0, 0)),
            pl.BlockSpec((F, F), lambda i: (0, 0)),
            pl.BlockSpec((1, F), lambda i: (0, 0)),
        ],
        out_specs=[
            pl.BlockSpec((_BE, HH), lambda i: (i, 0)),
            pl.BlockSpec((_BE, HH), lambda i: (i, 0)),
        ],
        out_shape=[
            jax.ShapeDtypeStruct((E, HH), jnp.float32),
            jax.ShapeDtypeStruct((E, HH), jnp.float32),
        ],
    )(edge_attr, edge_length2d, Wf1, bf1, Wf2, bf2)


def _sc_body(x1a_hbm, x1b_hbm, wfa_hbm, wfb_hbm, src_hbm, dst_hbm,
             outa_hbm, outb_hbm,
             src_v, dst_v, in_rows, wf_bufs, out_rows,
             agg_sh, x1_sh, sem_g, sem_w, sem_s, sem_i):
    cid = lax.axis_index("c")
    sid = lax.axis_index("s")

    zero16 = jnp.zeros((16,), jnp.float32)

    def zrow(i, carry):
        for j in range(HH // 16):
            out_rows[0][i, pl.ds(j * 16, 16)] = zero16
        return carry

    lax.fori_loop(0, CHUNK, zrow, 0)

    def zcp(k, carry):
        pltpu.sync_copy(out_rows[0], agg_sh.at[pl.ds(sid * RPT + k * CHUNK, CHUNK)])
        return carry

    lax.fori_loop(0, RPT // CHUNK, zcp, 0)

    def stage_x1(x1_hbm):
        for k in range(RPT // CHUNK):
            row0 = sid * RPT + k * CHUNK

            @pl.when(row0 < N)
            def _():
                pltpu.sync_copy(x1_hbm.at[pl.ds(row0, CHUNK)], in_rows[0])
                pltpu.sync_copy(in_rows[0], x1_sh.at[pl.ds(row0, CHUNK)])

    @pl.when(cid == 0)
    def _():
        stage_x1(x1a_hbm)

    @pl.when(cid == 1)
    def _():
        stage_x1(x1b_hbm)

    pltpu.sync_copy(src_hbm.at[sid, 0], src_v[0])
    pltpu.sync_copy(dst_hbm.at[sid, 0], dst_v[0])
    plsc.subcore_barrier()

    ebase = sid * EPT

    def run_half(wf_hbm):
        def issue_in(q, lc, b, g):
            pltpu.async_copy(x1_sh.at[src_v[q].at[lc]], in_rows[b], sem_g[b])
            pltpu.async_copy(wf_hbm.at[pl.ds(ebase + g * CHUNK, CHUNK)],
                             wf_bufs[b], sem_w[b])

        def wait_in(q, lc, b, g):
            pltpu.make_async_copy(x1_sh.at[src_v[q].at[lc]], in_rows[b],
                                  sem_g[b]).wait()
            pltpu.make_async_copy(wf_hbm.at[pl.ds(ebase + g * CHUNK, CHUNK)],
                                  wf_bufs[b], sem_w[b]).wait()

        def wait_scatter(b):
            pltpu.make_async_copy(out_rows[b], agg_sh.at[dst_v[0].at[0]],
                                  sem_s[b]).wait()

        def run_phase(p, q):
            @pl.when(p > 0)
            def _():
                wait_scatter(0)
                wait_scatter(1)
                pltpu.make_async_copy(src_hbm.at[sid, p], src_v[q],
                                      sem_i[0]).wait()
                pltpu.make_async_copy(dst_hbm.at[sid, p], dst_v[q],
                                      sem_i[1]).wait()

            @pl.when(p + 1 < NPHASE)
            def _():
                pltpu.async_copy(src_hbm.at[sid, p + 1], src_v[1 - q], sem_i[0])
                pltpu.async_copy(dst_hbm.at[sid, p + 1], dst_v[1 - q], sem_i[1])

            issue_in(q, 0, 0, p * PCHUNK + 0)
            issue_in(q, 1, 1, p * PCHUNK + 1)

            def pipe_body(go, c2):
                for b in range(2):
                    lc = go * 2 + b
                    gg = p * PCHUNK + lc
                    wait_in(q, lc, b, gg)

                    @pl.when(go >= 1)
                    def _():
                        wait_scatter(b)

                    def mul_row(i, c3):
                        for j in range(HH // 16):
                            sl = pl.ds(j * 16, 16)
                            out_rows[b][i, sl] = in_rows[b][i, sl] * wf_bufs[b][i, sl]
                        return c3

                    lax.fori_loop(0, CHUNK, mul_row, 0)
                    pltpu.async_copy(out_rows[b], agg_sh.at[dst_v[q].at[lc]],
                                     sem_s[b], add=True)

                    @pl.when(lc + 2 < PCHUNK)
                    def _():
                        issue_in(q, lc + 2, b, gg + 2)
                return c2

            lax.fori_loop(0, PCHUNK // 2, pipe_body, 0)

        def phase_pair(pp, carry):
            for qq in range(2):
                run_phase(pp * 2 + qq, qq)
            return carry

        lax.fori_loop(0, (NPHASE - 1) // 2, phase_pair, 0)
        run_phase(jnp.int32(NPHASE - 1), (NPHASE - 1) % 2)
        wait_scatter(0)
        wait_scatter(1)

    @pl.when(cid == 0)
    def _():
        run_half(wfa_hbm)

    @pl.when(cid == 1)
    def _():
        run_half(wfb_hbm)

    plsc.subcore_barrier()

    def dump_half(out_hbm):
        for k in range(RPT // CHUNK):
            pltpu.sync_copy(agg_sh.at[pl.ds(sid * RPT + k * CHUNK, CHUNK)],
                            out_rows[0])
            pltpu.sync_copy(out_rows[0], out_hbm.at[pl.ds(sid * RPT + k * CHUNK, CHUNK)])

    @pl.when(cid == 0)
    def _():
        dump_half(outa_hbm)

    @pl.when(cid == 1)
    def _():
        dump_half(outb_hbm)


def _sc_scatter(x1a, x1b, wfa, wfb, src, dst):
    mesh = plsc.VectorSubcoreMesh(core_axis_name="c", subcore_axis_name="s")
    fn = functools.partial(
        pl.kernel,
        mesh=mesh,
        out_type=[
            jax.ShapeDtypeStruct((NPAD, HH), jnp.float32),
            jax.ShapeDtypeStruct((NPAD, HH), jnp.float32),
        ],
        scratch_types=[
            [pltpu.VMEM((PCHUNK, CHUNK), jnp.int32) for _ in range(2)],
            [pltpu.VMEM((PCHUNK, CHUNK), jnp.int32) for _ in range(2)],
            [pltpu.VMEM((CHUNK, HH), jnp.float32) for _ in range(2)],
            [pltpu.VMEM((CHUNK, HH), jnp.float32) for _ in range(2)],
            [pltpu.VMEM((CHUNK, HH), jnp.float32) for _ in range(2)],
            pltpu.VMEM_SHARED((NPAD, HH), jnp.float32),
            pltpu.VMEM_SHARED((NPAD, HH), jnp.float32),
            [pltpu.SemaphoreType.DMA for _ in range(2)],
            [pltpu.SemaphoreType.DMA for _ in range(2)],
            [pltpu.SemaphoreType.DMA for _ in range(2)],
            [pltpu.SemaphoreType.DMA for _ in range(2)],
        ],
        compiler_params=pltpu.CompilerParams(use_tc_tiling_on_sc=False),
    )(_sc_body)
    return fn(x1a, x1b, wfa, wfb, src, dst)


def _out_body(pa_ref, pb_ref, wl2_ref, bl2_ref, wl_ref, bl_ref, o_ref):
    z = jnp.concatenate([pa_ref[...], pb_ref[...]], axis=1)
    x2 = jnp.dot(z, wl2_ref[...], preferred_element_type=jnp.float32) + bl2_ref[...]
    x3 = _fast_gelu(x2)
    o_ref[...] = jnp.dot(x3, wl_ref[...], preferred_element_type=jnp.float32) + bl_ref[...]


def _tc_out(pa, pb, Wl2, bl2, Wl, bl):
    return pl.pallas_call(
        _out_body,
        grid=(10,),
        in_specs=[
            pl.BlockSpec((N // 10, HH), lambda i: (i, 0)),
            pl.BlockSpec((N // 10, HH), lambda i: (i, 0)),
            pl.BlockSpec((F, H), lambda i: (0, 0)),
            pl.BlockSpec((1, H), lambda i: (0, 0)),
            pl.BlockSpec((H, H), lambda i: (0, 0)),
            pl.BlockSpec((1, H), lambda i: (0, 0)),
        ],
        out_specs=pl.BlockSpec((N // 10, H), lambda i: (i, 0)),
        out_shape=jax.ShapeDtypeStruct((N, H), jnp.float32),
    )(pa, pb, Wl2, bl2, Wl, bl)


def kernel(x, edge_index, edge_length, edge_attr, Wf1, bf1, Wf2, bf2,
           Wl1, Wl2, bl2, Wl, bl):
    x1a, x1b = _tc_x1(x, Wl1)
    wfa, wfb = _tc_filter(edge_attr, edge_length.reshape(E, 1), Wf1,
                          bf1.reshape(1, F), Wf2, bf2.reshape(1, F))
    src = edge_index[0].reshape(NS, NPHASE, PCHUNK, CHUNK)
    dst = edge_index[1].reshape(NS, NPHASE, PCHUNK, CHUNK)
    pa, pb = _sc_scatter(x1a, x1b, wfa, wfb, src, dst)
    out = _tc_out(pa[:N], pb[:N], Wl2, bl2.reshape(1, H),
                  Wl, bl.reshape(1, H))
    return out
